# baseline (speedup 1.0000x reference)
import jax
import jax.numpy as jnp
from jax.experimental import pallas as pl
from jax.experimental.pallas import tpu as pltpu

_B, _H, _W = 16, 512, 512
_NSHIFT = 29
_BF = jnp.bfloat16


def _fold_colsum(x16, folds):
    del folds
    ones = jnp.ones((1, x16.shape[0]), _BF)
    return jax.lax.dot_general(
        ones, x16, (((1,), (0,)), ((), ())),
        preferred_element_type=jnp.float32)


def _clamp01(t):
    return jnp.minimum(jnp.maximum(t, _BF(0.0)), _BF(1.0))


def _loss_kernel(pred_ref, mask_ref, out_ref, acc_ref):
    b = pl.program_id(0)
    p = jnp.round(pred_ref[0]).astype(_BF)
    mf = mask_ref[0].astype(_BF)
    pm = p * mf
    x = p + (_BF(100.0) - _BF(100.0) * mf)

    zeros = jnp.zeros((_H, _NSHIFT), _BF)
    acc = jnp.zeros((_H, _W), _BF)
    for s in range(1, _NSHIFT + 1):
        shifted = jnp.concatenate([pm[:, s:], zeros[:, :s]], axis=1)
        acc = acc + _clamp01(shifted - x)
    desc = jnp.sum(_fold_colsum(acc, 3))

    nmask = _fold_colsum(mf, 7)
    n = _fold_colsum(jnp.minimum(pm, _BF(1.0)), 7)
    zoff = float(_H) - n
    k_lo = jnp.floor((n + 1.0) * 0.5)
    k_hi = jnp.floor(n * 0.5) + 1.0

    lo = jnp.ones((1, _W), jnp.float32)
    hi = jnp.full((1, _W), 26.0, jnp.float32)
    for _ in range(5):
        mid = jnp.floor((lo + hi) * 0.5)
        thr = (mid + 1.0).astype(_BF)
        rank = _fold_colsum(_clamp01(thr - pm), 7) - zoff
        ge = rank >= k_lo
        hi = jnp.where(ge, mid, hi)
        lo = jnp.where(ge, lo, mid + 1.0)
    lower = lo
    rank_lower = _fold_colsum(_clamp01((lower + 1.0).astype(_BF) - pm), 7) - zoff
    same = rank_lower >= k_hi
    d = pm - lower.astype(_BF)
    eqi = jnp.maximum(_BF(1.0) - d * d, _BF(0.0))
    equal = _fold_colsum(eqi, 7)
    contrib = jnp.where(n > 0.0, nmask - jnp.where(same, equal, 0.0), 0.0)
    vert = jnp.sum(contrib)

    @pl.when(b == 0)
    def _init():
        acc_ref[0] = 0.0
        acc_ref[1] = 0.0

    acc_ref[0] += desc
    acc_ref[1] += vert

    @pl.when(b == _B - 1)
    def _fini():
        out_ref[0, 0] = (20.0 * acc_ref[0] + acc_ref[1]) / float(_B * _H * _W)


def _run(predictions, detection_mask):
    pred = predictions.reshape(_B, _H, _W)
    partials = pl.pallas_call(
        _loss_kernel,
        grid=(_B,),
        in_specs=[
            pl.BlockSpec((1, _H, _W), lambda b: (b, 0, 0)),
            pl.BlockSpec((1, _H, _W), lambda b: (b, 0, 0)),
        ],
        out_specs=pl.BlockSpec(memory_space=pltpu.SMEM),
        out_shape=jax.ShapeDtypeStruct((1, 1), jnp.float32),
        scratch_shapes=[pltpu.SMEM((2,), jnp.float32)],
    )(pred, detection_mask)
    return partials[0, 0]


def kernel(targets, predictions, detection_mask, weak_mask):
    return _run(predictions, detection_mask)


# speedup vs baseline: 1.0017x; 1.0017x over previous
import jax
import jax.numpy as jnp
from jax.experimental import pallas as pl
from jax.experimental.pallas import tpu as pltpu

_B, _H, _W = 16, 512, 512
_BB = 4
_NSTEP = _B // _BB
_NSHIFT = 29
_BF = jnp.bfloat16


def _fold_colsum(x16, folds):
    h = x16.shape[-2]
    for _ in range(folds):
        h //= 2
        x16 = x16[..., :h, :] + x16[..., h:, :]
    return jnp.sum(x16.astype(jnp.float32), axis=-2, keepdims=True)


def _clamp01(t):
    return jnp.minimum(jnp.maximum(t, _BF(0.0)), _BF(1.0))


def _loss_kernel(pred_ref, mask_ref, out_ref, acc_ref):
    b = pl.program_id(0)
    p = jnp.round(pred_ref[...]).astype(_BF)
    mf = mask_ref[...].astype(_BF)
    pm = p * mf
    x = p + (_BF(100.0) - _BF(100.0) * mf)

    zeros = jnp.zeros((_BB, _H, _NSHIFT), _BF)
    acc = jnp.zeros((_BB, _H, _W), _BF)
    for s in range(1, _NSHIFT + 1):
        shifted = jnp.concatenate([pm[..., s:], zeros[..., :s]], axis=2)
        acc = acc + _clamp01(shifted - x)
    desc = jnp.sum(_fold_colsum(acc.reshape(_BB * _H, _W), 3))

    nmask = _fold_colsum(mf, 7)
    n = _fold_colsum(jnp.minimum(pm, _BF(1.0)), 7)
    zoff = float(_H) - n
    k_lo = jnp.floor((n + 1.0) * 0.5)
    k_hi = jnp.floor(n * 0.5) + 1.0

    lo = jnp.ones((_BB, 1, _W), jnp.float32)
    hi = jnp.full((_BB, 1, _W), 26.0, jnp.float32)
    for _ in range(5):
        mid = jnp.floor((lo + hi) * 0.5)
        thr = (mid + 1.0).astype(_BF)
        rank = _fold_colsum(_clamp01(thr - pm), 7) - zoff
        ge = rank >= k_lo
        hi = jnp.where(ge, mid, hi)
        lo = jnp.where(ge, lo, mid + 1.0)
    lower = lo
    rank_lower = _fold_colsum(_clamp01((lower + 1.0).astype(_BF) - pm), 7) - zoff
    same = rank_lower >= k_hi
    d = pm - lower.astype(_BF)
    eqi = jnp.maximum(_BF(1.0) - d * d, _BF(0.0))
    equal = _fold_colsum(eqi, 7)
    contrib = jnp.where(n > 0.0, nmask - jnp.where(same, equal, 0.0), 0.0)
    vert = jnp.sum(contrib)

    @pl.when(b == 0)
    def _init():
        acc_ref[0] = 0.0
        acc_ref[1] = 0.0

    acc_ref[0] += desc
    acc_ref[1] += vert

    @pl.when(b == _NSTEP - 1)
    def _fini():
        out_ref[0, 0] = (20.0 * acc_ref[0] + acc_ref[1]) / float(_B * _H * _W)


def _run(predictions, detection_mask):
    pred = predictions.reshape(_B, _H, _W)
    partials = pl.pallas_call(
        _loss_kernel,
        grid=(_NSTEP,),
        in_specs=[
            pl.BlockSpec((_BB, _H, _W), lambda b: (b, 0, 0)),
            pl.BlockSpec((_BB, _H, _W), lambda b: (b, 0, 0)),
        ],
        out_specs=pl.BlockSpec(memory_space=pltpu.SMEM),
        out_shape=jax.ShapeDtypeStruct((1, 1), jnp.float32),
        scratch_shapes=[pltpu.SMEM((2,), jnp.float32)],
    )(pred, detection_mask)
    return partials[0, 0]


def kernel(targets, predictions, detection_mask, weak_mask):
    return _run(predictions, detection_mask)


# speedup vs baseline: 1.0256x; 1.0239x over previous
import jax
import jax.numpy as jnp
from jax.experimental import pallas as pl
from jax.experimental.pallas import tpu as pltpu

_B, _H, _W = 16, 512, 512
_BB = 2
_NSTEP = _B // _BB
_NSHIFT = 29
_BF = jnp.bfloat16


def _fold_colsum(x16, folds):
    h = x16.shape[-2]
    for _ in range(folds):
        h //= 2
        x16 = x16[..., :h, :] + x16[..., h:, :]
    return jnp.sum(x16.astype(jnp.float32), axis=-2, keepdims=True)


def _clamp01(t):
    return jnp.minimum(jnp.maximum(t, _BF(0.0)), _BF(1.0))


def _loss_kernel(pred_ref, mask_ref, out_ref, acc_ref):
    b = pl.program_id(0)
    p = jnp.round(pred_ref[...]).astype(_BF)
    mf = mask_ref[...].astype(_BF)
    pm = p * mf
    x = p + (_BF(100.0) - _BF(100.0) * mf)

    zeros = jnp.zeros((_BB, _H, _NSHIFT), _BF)
    acc = jnp.zeros((_BB, _H, _W), _BF)
    for s in range(1, _NSHIFT + 1):
        shifted = jnp.concatenate([pm[..., s:], zeros[..., :s]], axis=2)
        acc = acc + _clamp01(shifted - x)
    desc = jnp.sum(_fold_colsum(acc.reshape(_BB * _H, _W), 3))

    nmask = _fold_colsum(mf, 7)
    n = _fold_colsum(jnp.minimum(pm, _BF(1.0)), 7)
    zoff = float(_H) - n
    k_lo = jnp.floor((n + 1.0) * 0.5)
    k_hi = jnp.floor(n * 0.5) + 1.0

    lo = jnp.ones((_BB, 1, _W), jnp.float32)
    hi = jnp.full((_BB, 1, _W), 26.0, jnp.float32)
    for _ in range(5):
        mid = jnp.floor((lo + hi) * 0.5)
        thr = (mid + 1.0).astype(_BF)
        rank = _fold_colsum(_clamp01(thr - pm), 7) - zoff
        ge = rank >= k_lo
        hi = jnp.where(ge, mid, hi)
        lo = jnp.where(ge, lo, mid + 1.0)
    lower = lo
    rank_lower = _fold_colsum(_clamp01((lower + 1.0).astype(_BF) - pm), 7) - zoff
    same = rank_lower >= k_hi
    d = pm - lower.astype(_BF)
    eqi = jnp.maximum(_BF(1.0) - d * d, _BF(0.0))
    equal = _fold_colsum(eqi, 7)
    contrib = jnp.where(n > 0.0, nmask - jnp.where(same, equal, 0.0), 0.0)
    vert = jnp.sum(contrib)

    @pl.when(b == 0)
    def _init():
        acc_ref[0] = 0.0
        acc_ref[1] = 0.0

    acc_ref[0] += desc
    acc_ref[1] += vert

    @pl.when(b == _NSTEP - 1)
    def _fini():
        out_ref[0, 0] = (20.0 * acc_ref[0] + acc_ref[1]) / float(_B * _H * _W)


def _run(predictions, detection_mask):
    pred = predictions.reshape(_B, _H, _W)
    partials = pl.pallas_call(
        _loss_kernel,
        grid=(_NSTEP,),
        in_specs=[
            pl.BlockSpec((_BB, _H, _W), lambda b: (b, 0, 0)),
            pl.BlockSpec((_BB, _H, _W), lambda b: (b, 0, 0)),
        ],
        out_specs=pl.BlockSpec(memory_space=pltpu.SMEM),
        out_shape=jax.ShapeDtypeStruct((1, 1), jnp.float32),
        scratch_shapes=[pltpu.SMEM((2,), jnp.float32)],
    )(pred, detection_mask)
    return partials[0, 0]


def kernel(targets, predictions, detection_mask, weak_mask):
    return _run(predictions, detection_mask)


# speedup vs baseline: 1.0382x; 1.0123x over previous
import jax
import jax.numpy as jnp
from jax.experimental import pallas as pl
from jax.experimental.pallas import tpu as pltpu

_B, _H, _W = 16, 512, 512
_BB = 1
_NSTEP = _B // _BB
_NSHIFT = 29
_BF = jnp.bfloat16


def _fold_colsum(x16, folds):
    h = x16.shape[-2]
    for _ in range(folds):
        h //= 2
        x16 = x16[..., :h, :] + x16[..., h:, :]
    return jnp.sum(x16.astype(jnp.float32), axis=-2, keepdims=True)


def _clamp01(t):
    return jnp.minimum(jnp.maximum(t, _BF(0.0)), _BF(1.0))


def _loss_kernel(pred_ref, mask_ref, out_ref, acc_ref):
    b = pl.program_id(0)
    p = jnp.round(pred_ref[...]).astype(_BF)
    mf = mask_ref[...].astype(_BF)
    pm = p * mf
    x = p + (_BF(100.0) - _BF(100.0) * mf)

    zeros = jnp.zeros((_BB, _H, _NSHIFT), _BF)
    acc = jnp.zeros((_BB, _H, _W), _BF)
    for s in range(1, _NSHIFT + 1):
        shifted = jnp.concatenate([pm[..., s:], zeros[..., :s]], axis=2)
        acc = acc + _clamp01(shifted - x)
    desc = jnp.sum(_fold_colsum(acc.reshape(_BB * _H, _W), 3))

    nmask = _fold_colsum(mf, 7)
    n = _fold_colsum(jnp.minimum(pm, _BF(1.0)), 7)
    zoff = float(_H) - n
    k_lo = jnp.floor((n + 1.0) * 0.5)
    k_hi = jnp.floor(n * 0.5) + 1.0

    lo = jnp.ones((_BB, 1, _W), jnp.float32)
    hi = jnp.full((_BB, 1, _W), 26.0, jnp.float32)
    for _ in range(5):
        mid = jnp.floor((lo + hi) * 0.5)
        thr = (mid + 1.0).astype(_BF)
        rank = _fold_colsum(_clamp01(thr - pm), 7) - zoff
        ge = rank >= k_lo
        hi = jnp.where(ge, mid, hi)
        lo = jnp.where(ge, lo, mid + 1.0)
    lower = lo
    rank_lower = _fold_colsum(_clamp01((lower + 1.0).astype(_BF) - pm), 7) - zoff
    same = rank_lower >= k_hi
    d = pm - lower.astype(_BF)
    eqi = jnp.maximum(_BF(1.0) - d * d, _BF(0.0))
    equal = _fold_colsum(eqi, 7)
    contrib = jnp.where(n > 0.0, nmask - jnp.where(same, equal, 0.0), 0.0)
    vert = jnp.sum(contrib)

    @pl.when(b == 0)
    def _init():
        acc_ref[0] = 0.0
        acc_ref[1] = 0.0

    acc_ref[0] += desc
    acc_ref[1] += vert

    @pl.when(b == _NSTEP - 1)
    def _fini():
        out_ref[0, 0] = (20.0 * acc_ref[0] + acc_ref[1]) / float(_B * _H * _W)


def _run(predictions, detection_mask):
    pred = predictions.reshape(_B, _H, _W)
    partials = pl.pallas_call(
        _loss_kernel,
        grid=(_NSTEP,),
        in_specs=[
            pl.BlockSpec((_BB, _H, _W), lambda b: (b, 0, 0)),
            pl.BlockSpec((_BB, _H, _W), lambda b: (b, 0, 0)),
        ],
        out_specs=pl.BlockSpec(memory_space=pltpu.SMEM),
        out_shape=jax.ShapeDtypeStruct((1, 1), jnp.float32),
        scratch_shapes=[pltpu.SMEM((2,), jnp.float32)],
    )(pred, detection_mask)
    return partials[0, 0]


def kernel(targets, predictions, detection_mask, weak_mask):
    return _run(predictions, detection_mask)
